# baseline (speedup 1.0000x reference)
import functools

import jax
import jax.numpy as jnp
from jax import lax
from jax.experimental import pallas as pl
from jax.experimental.pallas import tpu as pltpu
from jax.experimental.pallas import tpu_sc as plsc

H = 128
H3 = 384
N_NODES = 10000
N_EDGES = 320000

NC = 2
NS = 16
NW = NC * NS

EB = 128
E_PAD = 327680
NBLK = E_PAD // EB
GATHER_BLKS = NBLK // NW
NCHUNK = 4
CBLK = NBLK // NCHUNK
E_C = E_PAD // NCHUNK
SCATTER_BLKS = CBLK // NS
N_ACC = 10240
ROWS_PER_TILE = N_ACC // NS


def _phi_body(s_ref, w1_ref, b1_ref, w2_ref, b2_ref, o_ref):
    h = jax.nn.silu(jnp.dot(s_ref[...], w1_ref[...],
                            preferred_element_type=jnp.float32) + b1_ref[...])
    o_ref[...] = jnp.dot(h, w2_ref[...],
                         preferred_element_type=jnp.float32) + b2_ref[...]


def _phi_tc(s, W1, b1, W2, b2):
    blk = 1000 if N_NODES % 1000 == 0 else N_NODES
    grid = (N_NODES // blk,)
    return pl.pallas_call(
        _phi_body,
        grid=grid,
        in_specs=[
            pl.BlockSpec((blk, H), lambda i: (i, 0)),
            pl.BlockSpec((H, H), lambda i: (0, 0)),
            pl.BlockSpec((1, H), lambda i: (0, 0)),
            pl.BlockSpec((H, H3), lambda i: (0, 0)),
            pl.BlockSpec((1, H3), lambda i: (0, 0)),
        ],
        out_specs=pl.BlockSpec((blk, H3), lambda i: (i, 0)),
        out_shape=jax.ShapeDtypeStruct((N_NODES, H3), jnp.float32),
    )(s, W1, b1, W2, b2)


def _gather_pass(table_hbm, out_hbm, idx_all, r0, r1, g0, g1, w0, w1, base,
                 nblk, blk_off):
    def gather(b, rows, sem):
        return pltpu.async_copy(table_hbm.at[idx_all.at[b]], rows, sem)

    def wb(b, rows, sem):
        return pltpu.async_copy(
            rows, out_hbm.at[pl.ds((base - blk_off + b) * EB, EB)], sem)

    gather(0, r0, g0)
    gather(1, r1, g1)

    def body(i, carry):
        b = 2 * i
        pltpu.make_async_copy(table_hbm.at[idx_all.at[0]], r0, g0).wait()
        wb(b, r0, w0)
        pltpu.make_async_copy(table_hbm.at[idx_all.at[0]], r1, g1).wait()
        wb(b + 1, r1, w1)
        pltpu.make_async_copy(r0, out_hbm.at[pl.ds(base * EB, EB)], w0).wait()
        gather(b + 2, r0, g0)
        pltpu.make_async_copy(r1, out_hbm.at[pl.ds(base * EB, EB)], w1).wait()
        gather(b + 3, r1, g1)
        return carry

    lax.fori_loop(0, nblk // 2 - 1, body, 0)
    last = nblk - 2
    pltpu.make_async_copy(table_hbm.at[idx_all.at[0]], r0, g0).wait()
    wb(last, r0, w0)
    pltpu.make_async_copy(table_hbm.at[idx_all.at[0]], r1, g1).wait()
    wb(last + 1, r1, w1)
    pltpu.make_async_copy(r0, out_hbm.at[pl.ds(base * EB, EB)], w0).wait()
    pltpu.make_async_copy(r1, out_hbm.at[pl.ds(base * EB, EB)], w1).wait()


GBLK0 = 24
GBLK1 = 16


def _take16(vec, idx):
    return lax.gather(
        vec, idx[:, None],
        lax.GatherDimensionNumbers(offset_dims=(), collapsed_slice_dims=(0,),
                                   start_index_map=(0,)),
        slice_sizes=(1,),
        mode=lax.GatherScatterMode.PROMISE_IN_BOUNDS)


def _inner_pass(v_hbm, vec_hbm, inner_hbm, idx_all, r0, r1, vb0, vb1, ibuf,
                g0, g1, w0, base, nblk, blk_off):
    def gather(b, rows, vb, sem):
        pltpu.async_copy(v_hbm.at[idx_all.at[b]], rows, sem)
        pltpu.async_copy(vec_hbm.at[:, pl.ds((base + b) * EB, EB)], vb, sem)

    def gather_wait(rows, vb, sem):
        pltpu.make_async_copy(v_hbm.at[idx_all.at[0]], rows, sem).wait()
        pltpu.make_async_copy(vec_hbm.at[:, pl.ds(base * EB, EB)], vb,
                              sem).wait()

    def compute(rows, vb):
        def edge(e, carry):
            z = jnp.zeros((16,), jnp.int32)
            chunk = (e // 16) * 16
            lane = z + (e % 16)
            c0 = vb[0, pl.ds(chunk, 16)]
            c1 = vb[1, pl.ds(chunk, 16)]
            c2 = vb[2, pl.ds(chunk, 16)]
            s0 = _take16(c0, lane)
            s1 = _take16(c1, lane)
            s2 = _take16(c2, lane)
            for q in range(H // 16):
                o = q * 16
                ibuf[e, pl.ds(o, 16)] = (
                    s0 * rows[e, pl.ds(o, 16)]
                    + s1 * rows[e, pl.ds(H + o, 16)]
                    + s2 * rows[e, pl.ds(2 * H + o, 16)])
            return carry
        lax.fori_loop(0, EB, edge, 0)

    def step(b, rows, vb, sem):
        gather_wait(rows, vb, sem)
        compute(rows, vb)
        pltpu.sync_copy(ibuf,
                        inner_hbm.at[pl.ds((base - blk_off + b) * EB, EB)])
        return rows

    gather(0, r0, vb0, g0)
    gather(1, r1, vb1, g1)

    def body(i, carry):
        b = 2 * i
        step(b, r0, vb0, g0)
        gather(b + 2, r0, vb0, g0)
        step(b + 1, r1, vb1, g1)
        gather(b + 3, r1, vb1, g1)
        return carry

    lax.fori_loop(0, nblk // 2 - 1, body, 0)
    last = nblk - 2
    step(last, r0, vb0, g0)
    step(last + 1, r1, vb1, g1)


def _gather_body(blk_off, phi_hbm, v_hbm, vec_hbm, j_hbm, phij_hbm,
                 inner_hbm, idx_all, r0, r1, vb0, vb1, ibuf, g0, g1, w0, w1):
    c = lax.axis_index("c")
    sid = lax.axis_index("s")
    nblk = jnp.where(c == 0, GBLK0, GBLK1)
    base = blk_off + jnp.where(c == 0, sid * GBLK0,
                               NS * GBLK0 + sid * GBLK1)
    pltpu.sync_copy(j_hbm.at[pl.ds(base, GBLK1)],
                    idx_all.at[pl.ds(0, GBLK1)])

    @pl.when(c == 0)
    def _load_rest():
        pltpu.sync_copy(j_hbm.at[pl.ds(base + GBLK1, GBLK0 - GBLK1)],
                        idx_all.at[pl.ds(GBLK1, GBLK0 - GBLK1)])
    _gather_pass(phi_hbm, phij_hbm, idx_all, r0, r1, g0, g1, w0, w1, base,
                 nblk, blk_off)
    _inner_pass(v_hbm, vec_hbm, inner_hbm, idx_all, r0, r1, vb0, vb1, ibuf,
                g0, g1, w0, base, nblk, blk_off)


def _gather_sc(phi, v2d, j2d, vecT, chunk):
    mesh = plsc.VectorSubcoreMesh(core_axis_name="c", subcore_axis_name="s",
                                  num_cores=NC, num_subcores=NS)
    k = pl.kernel(
        functools.partial(_gather_body, chunk * CBLK),
        out_type=[
            jax.ShapeDtypeStruct((E_C, H3), jnp.float32),
            jax.ShapeDtypeStruct((E_C, H), jnp.float32),
        ],
        mesh=mesh,
        scratch_types=[
            pltpu.VMEM((GBLK0, EB), jnp.int32),
            pltpu.VMEM((EB, H3), jnp.float32),
            pltpu.VMEM((EB, H3), jnp.float32),
            pltpu.VMEM((3, EB), jnp.float32),
            pltpu.VMEM((3, EB), jnp.float32),
            pltpu.VMEM((EB, H), jnp.float32),
            pltpu.SemaphoreType.DMA,
            pltpu.SemaphoreType.DMA,
            pltpu.SemaphoreType.DMA,
            pltpu.SemaphoreType.DMA,
        ],
    )
    return k(phi, v2d, vecT, j2d)


def _edge_body(rbf_ref, wr_ref, br_ref, cut_ref, vx_ref, vy_ref, vz_ref,
               phij_ref, inner_ref, y_ref):
    w = (jnp.dot(rbf_ref[...], wr_ref[...],
                 preferred_element_type=jnp.float32) + br_ref[...]) * cut_ref[...]
    x = phij_ref[...] * w
    x_ss = x[:, 0:H]
    x_sv = x[:, H:2 * H]
    x_vv = x[:, 2 * H:3 * H]
    vx, vy, vz = vx_ref[...], vy_ref[...], vz_ref[...]
    coeff = x_sv + inner_ref[...] * x_vv
    y_ref[...] = jnp.concatenate(
        [x_ss, coeff * vx, coeff * vy, coeff * vz], axis=1)


def _edge_tc(rbf, Wr, br, cut, vx, vy, vz, phij, inner, chunk):
    blk = 512
    grid = (E_C // blk,)
    koff = chunk * (E_C // blk)
    return pl.pallas_call(
        _edge_body,
        grid=grid,
        in_specs=[
            pl.BlockSpec((blk, 20), lambda i: (i + koff, 0)),
            pl.BlockSpec((20, H3), lambda i: (0, 0)),
            pl.BlockSpec((1, H3), lambda i: (0, 0)),
            pl.BlockSpec((blk, 1), lambda i: (i + koff, 0)),
            pl.BlockSpec((blk, 1), lambda i: (i + koff, 0)),
            pl.BlockSpec((blk, 1), lambda i: (i + koff, 0)),
            pl.BlockSpec((blk, 1), lambda i: (i + koff, 0)),
            pl.BlockSpec((blk, H3), lambda i: (i, 0)),
            pl.BlockSpec((blk, H), lambda i: (i, 0)),
        ],
        out_specs=pl.BlockSpec((blk, 4 * H), lambda i: (i, 0)),
        out_shape=jax.ShapeDtypeStruct((E_C, 4 * H), jnp.float32),
    )(rbf, Wr, br, cut, vx, vy, vz, phij, inner)


def _scatter_body(blk_off, y_hbm, i_hbm, init_hbm, acc_hbm, acc_sh,
                  i0, i1, yb0, yb1, rs0, rs1, ss0, ss1):
    c = lax.axis_index("c")
    sid = lax.axis_index("s")
    r0 = sid * ROWS_PER_TILE
    base = sid * SCATTER_BLKS

    for g in range(2):
        grp = c * 2 + g
        pltpu.sync_copy(init_hbm.at[grp, pl.ds(r0, ROWS_PER_TILE)],
                        acc_sh.at[pl.ds(r0, ROWS_PER_TILE)])
        plsc.subcore_barrier()

        def rd(b, ib, yb, sem):
            pltpu.async_copy(i_hbm.at[blk_off + base + b], ib, sem)
            pltpu.async_copy(
                y_hbm.at[pl.ds((base + b) * EB, EB), pl.ds(grp * H, H)],
                yb, sem)

        def rd_wait(ib, yb, sem):
            pltpu.make_async_copy(i_hbm.at[blk_off + base], ib, sem).wait()
            pltpu.make_async_copy(y_hbm.at[pl.ds(base * EB, EB),
                                           pl.ds(grp * H, H)], yb, sem).wait()

        def sc(ib, yb, sem):
            pltpu.async_copy(yb, acc_sh.at[ib], sem, add=True)

        def sc_wait(ib, yb, sem):
            pltpu.make_async_copy(yb, acc_sh.at[ib], sem).wait()

        rd(0, i0, yb0, rs0)
        rd(1, i1, yb1, rs1)

        def body(i, carry):
            b = 2 * i
            rd_wait(i0, yb0, rs0)
            sc(i0, yb0, ss0)
            rd_wait(i1, yb1, rs1)
            sc(i1, yb1, ss1)
            sc_wait(i0, yb0, ss0)
            rd(b + 2, i0, yb0, rs0)
            sc_wait(i1, yb1, ss1)
            rd(b + 3, i1, yb1, rs1)
            return carry

        lax.fori_loop(0, SCATTER_BLKS // 2 - 1, body, 0)
        rd_wait(i0, yb0, rs0)
        sc(i0, yb0, ss0)
        rd_wait(i1, yb1, rs1)
        sc(i1, yb1, ss1)
        sc_wait(i0, yb0, ss0)
        sc_wait(i1, yb1, ss1)

        plsc.subcore_barrier()
        pltpu.sync_copy(acc_sh.at[pl.ds(r0, ROWS_PER_TILE)],
                        acc_hbm.at[grp, pl.ds(r0, ROWS_PER_TILE)])
        plsc.subcore_barrier()


def _scatter_sc(y, i2d, init, chunk):
    mesh = plsc.VectorSubcoreMesh(core_axis_name="c", subcore_axis_name="s",
                                  num_cores=NC, num_subcores=NS)
    k = pl.kernel(
        functools.partial(_scatter_body, chunk * CBLK),
        out_type=[jax.ShapeDtypeStruct((4, N_ACC, H), jnp.float32)],
        mesh=mesh,
        scratch_types=[
            pltpu.VMEM_SHARED((N_ACC, H), jnp.float32),
            pltpu.VMEM((EB,), jnp.int32),
            pltpu.VMEM((EB,), jnp.int32),
            pltpu.VMEM((EB, H), jnp.float32),
            pltpu.VMEM((EB, H), jnp.float32),
            pltpu.SemaphoreType.DMA,
            pltpu.SemaphoreType.DMA,
            pltpu.SemaphoreType.DMA,
            pltpu.SemaphoreType.DMA,
        ],
    )
    return k(y, i2d, init)[0]


def _update_body(s_ref, v_ref, acc_ref, wv_ref, wu1_ref, bu1_ref, wu2_ref,
                 bu2_ref, snew_ref, vnew_ref):
    s2 = s_ref[...] + acc_ref[0]
    v = v_ref[...]
    v2_0 = v[:, 0:H] + acc_ref[1]
    v2_1 = v[:, H:2 * H] + acc_ref[2]
    v2_2 = v[:, 2 * H:3 * H] + acc_ref[3]
    wv = wv_ref[...]
    vp0 = jnp.dot(v2_0, wv, preferred_element_type=jnp.float32)
    vp1 = jnp.dot(v2_1, wv, preferred_element_type=jnp.float32)
    vp2 = jnp.dot(v2_2, wv, preferred_element_type=jnp.float32)
    vu0, vv0 = vp0[:, 0:H], vp0[:, H:2 * H]
    vu1, vv1 = vp1[:, 0:H], vp1[:, H:2 * H]
    vu2, vv2 = vp2[:, 0:H], vp2[:, H:2 * H]
    norm = jnp.sqrt(vv0 * vv0 + vv1 * vv1 + vv2 * vv2)
    s_in = jnp.concatenate([s2, norm], axis=1)
    hid = jax.nn.silu(jnp.dot(s_in, wu1_ref[...],
                              preferred_element_type=jnp.float32) + bu1_ref[...])
    s_out = jnp.dot(hid, wu2_ref[...],
                    preferred_element_type=jnp.float32) + bu2_ref[...]
    a_ss = s_out[:, 0:H]
    a_sv = s_out[:, H:2 * H]
    a_vv = s_out[:, 2 * H:3 * H]
    snew_ref[...] = s2 + a_ss
    vnew_ref[...] = jnp.concatenate([
        v2_0 + a_vv * vu0 + a_sv * vv0,
        v2_1 + a_vv * vu1 + a_sv * vv1,
        v2_2 + a_vv * vu2 + a_sv * vv2,
    ], axis=1)


def _update_tc(s, v2d, acc, Wv, Wu1, bu1, Wu2, bu2):
    blk = 1000 if N_NODES % 1000 == 0 else N_NODES
    grid = (N_NODES // blk,)
    return pl.pallas_call(
        _update_body,
        grid=grid,
        in_specs=[
            pl.BlockSpec((blk, H), lambda i: (i, 0)),
            pl.BlockSpec((blk, H3), lambda i: (i, 0)),
            pl.BlockSpec((4, blk, H), lambda i: (0, i, 0)),
            pl.BlockSpec((H, 2 * H), lambda i: (0, 0)),
            pl.BlockSpec((2 * H, H), lambda i: (0, 0)),
            pl.BlockSpec((1, H), lambda i: (0, 0)),
            pl.BlockSpec((H, H3), lambda i: (0, 0)),
            pl.BlockSpec((1, H3), lambda i: (0, 0)),
        ],
        out_specs=[
            pl.BlockSpec((blk, H), lambda i: (i, 0)),
            pl.BlockSpec((blk, H3), lambda i: (i, 0)),
        ],
        out_shape=[
            jax.ShapeDtypeStruct((N_NODES, H), jnp.float32),
            jax.ShapeDtypeStruct((N_NODES, H3), jnp.float32),
        ],
    )(s, v2d, acc, Wv, Wu1, bu1, Wu2, bu2)


def kernel(s, v, edge_index, edge_rbf, edge_cutoff, edge_vec,
           W1, b1, W2, b2, Wr, br, Wv, Wu1, bu1, Wu2, bu2):
    pad = E_PAD - N_EDGES
    j_idx = edge_index[1].astype(jnp.int32)
    i_idx = edge_index[0].astype(jnp.int32)
    j2d = jnp.pad(j_idx, (0, pad)).reshape(NBLK, EB)
    i2d = jnp.pad(i_idx, (0, pad)).reshape(NBLK, EB)
    rbf_p = jnp.pad(edge_rbf, ((0, pad), (0, 0)))
    cut_p = jnp.pad(edge_cutoff, (0, pad)).reshape(E_PAD, 1)
    vec_p = jnp.pad(edge_vec, ((0, pad), (0, 0)))
    vx = vec_p[:, 0:1]
    vy = vec_p[:, 1:2]
    vz = vec_p[:, 2:3]
    v2d = v.reshape(N_NODES, H3)

    vecT = vec_p.T
    phi = _phi_tc(s, W1, b1.reshape(1, H), W2, b2.reshape(1, H3))
    acc = jnp.zeros((4, N_ACC, H), jnp.float32)
    for k in range(NCHUNK):
        phij, inner = _gather_sc(phi, v2d, j2d, vecT, k)
        y = _edge_tc(rbf_p, Wr, br.reshape(1, H3), cut_p, vx, vy, vz, phij,
                     inner, k)
        acc = _scatter_sc(y, i2d, acc, k)
    s_new, vnew2d = _update_tc(s, v2d, acc, Wv, Wu1, bu1.reshape(1, H),
                               Wu2, bu2.reshape(1, H3))
    return (s_new, vnew2d.reshape(N_NODES, 3, H))


# speedup vs baseline: 1.0153x; 1.0153x over previous
import functools

import jax
import jax.numpy as jnp
from jax import lax
from jax.experimental import pallas as pl
from jax.experimental.pallas import tpu as pltpu
from jax.experimental.pallas import tpu_sc as plsc

H = 128
H3 = 384
N_NODES = 10000
N_EDGES = 320000

NC = 2
NS = 16
NW = NC * NS

EB = 128
E_PAD = 327680
NBLK = E_PAD // EB
GATHER_BLKS = NBLK // NW
NCHUNK = 2
CBLK = NBLK // NCHUNK
E_C = E_PAD // NCHUNK
SCATTER_BLKS = CBLK // NS
N_ACC = 10240
ROWS_PER_TILE = N_ACC // NS


def _phi_body(s_ref, w1_ref, b1_ref, w2_ref, b2_ref, o_ref):
    h = jax.nn.silu(jnp.dot(s_ref[...], w1_ref[...],
                            preferred_element_type=jnp.float32) + b1_ref[...])
    o_ref[...] = jnp.dot(h, w2_ref[...],
                         preferred_element_type=jnp.float32) + b2_ref[...]


def _phi_tc(s, W1, b1, W2, b2):
    blk = 1000 if N_NODES % 1000 == 0 else N_NODES
    grid = (N_NODES // blk,)
    return pl.pallas_call(
        _phi_body,
        grid=grid,
        in_specs=[
            pl.BlockSpec((blk, H), lambda i: (i, 0)),
            pl.BlockSpec((H, H), lambda i: (0, 0)),
            pl.BlockSpec((1, H), lambda i: (0, 0)),
            pl.BlockSpec((H, H3), lambda i: (0, 0)),
            pl.BlockSpec((1, H3), lambda i: (0, 0)),
        ],
        out_specs=pl.BlockSpec((blk, H3), lambda i: (i, 0)),
        out_shape=jax.ShapeDtypeStruct((N_NODES, H3), jnp.float32),
    )(s, W1, b1, W2, b2)


def _gather_pass(table_hbm, out_hbm, idx_all, r0, r1, g0, g1, w0, w1, base,
                 nblk, blk_off):
    def gather(b, rows, sem):
        return pltpu.async_copy(table_hbm.at[idx_all.at[b]], rows, sem)

    def wb(b, rows, sem):
        return pltpu.async_copy(
            rows, out_hbm.at[pl.ds((base - blk_off + b) * EB, EB)], sem)

    gather(0, r0, g0)
    gather(1, r1, g1)

    def body(i, carry):
        b = 2 * i
        pltpu.make_async_copy(table_hbm.at[idx_all.at[0]], r0, g0).wait()
        wb(b, r0, w0)
        pltpu.make_async_copy(table_hbm.at[idx_all.at[0]], r1, g1).wait()
        wb(b + 1, r1, w1)
        pltpu.make_async_copy(r0, out_hbm.at[pl.ds(base * EB, EB)], w0).wait()
        gather(b + 2, r0, g0)
        pltpu.make_async_copy(r1, out_hbm.at[pl.ds(base * EB, EB)], w1).wait()
        gather(b + 3, r1, g1)
        return carry

    lax.fori_loop(0, nblk // 2 - 1, body, 0)
    last = nblk - 2
    pltpu.make_async_copy(table_hbm.at[idx_all.at[0]], r0, g0).wait()
    wb(last, r0, w0)
    pltpu.make_async_copy(table_hbm.at[idx_all.at[0]], r1, g1).wait()
    wb(last + 1, r1, w1)
    pltpu.make_async_copy(r0, out_hbm.at[pl.ds(base * EB, EB)], w0).wait()
    pltpu.make_async_copy(r1, out_hbm.at[pl.ds(base * EB, EB)], w1).wait()


GBLK0 = 48
GBLK1 = 32


def _take16(vec, idx):
    return lax.gather(
        vec, idx[:, None],
        lax.GatherDimensionNumbers(offset_dims=(), collapsed_slice_dims=(0,),
                                   start_index_map=(0,)),
        slice_sizes=(1,),
        mode=lax.GatherScatterMode.PROMISE_IN_BOUNDS)


def _inner_pass(v_hbm, vec_hbm, inner_hbm, idx_all, r0, r1, vb0, vb1, ibuf,
                g0, g1, w0, base, nblk, blk_off):
    def gather(b, rows, vb, sem):
        pltpu.async_copy(v_hbm.at[idx_all.at[b]], rows, sem)
        pltpu.async_copy(vec_hbm.at[:, pl.ds((base + b) * EB, EB)], vb, sem)

    def gather_wait(rows, vb, sem):
        pltpu.make_async_copy(v_hbm.at[idx_all.at[0]], rows, sem).wait()
        pltpu.make_async_copy(vec_hbm.at[:, pl.ds(base * EB, EB)], vb,
                              sem).wait()

    def compute(rows, vb):
        def edge(e, carry):
            z = jnp.zeros((16,), jnp.int32)
            chunk = (e // 16) * 16
            lane = z + (e % 16)
            c0 = vb[0, pl.ds(chunk, 16)]
            c1 = vb[1, pl.ds(chunk, 16)]
            c2 = vb[2, pl.ds(chunk, 16)]
            s0 = _take16(c0, lane)
            s1 = _take16(c1, lane)
            s2 = _take16(c2, lane)
            for q in range(H // 16):
                o = q * 16
                ibuf[e, pl.ds(o, 16)] = (
                    s0 * rows[e, pl.ds(o, 16)]
                    + s1 * rows[e, pl.ds(H + o, 16)]
                    + s2 * rows[e, pl.ds(2 * H + o, 16)])
            return carry
        lax.fori_loop(0, EB, edge, 0)

    def step(b, rows, vb, sem):
        gather_wait(rows, vb, sem)
        compute(rows, vb)
        pltpu.sync_copy(ibuf,
                        inner_hbm.at[pl.ds((base - blk_off + b) * EB, EB)])
        return rows

    gather(0, r0, vb0, g0)
    gather(1, r1, vb1, g1)

    def body(i, carry):
        b = 2 * i
        step(b, r0, vb0, g0)
        gather(b + 2, r0, vb0, g0)
        step(b + 1, r1, vb1, g1)
        gather(b + 3, r1, vb1, g1)
        return carry

    lax.fori_loop(0, nblk // 2 - 1, body, 0)
    last = nblk - 2
    step(last, r0, vb0, g0)
    step(last + 1, r1, vb1, g1)


def _gather_body(blk_off, phi_hbm, v_hbm, vec_hbm, j_hbm, phij_hbm,
                 inner_hbm, idx_all, r0, r1, vb0, vb1, ibuf, g0, g1, w0, w1):
    c = lax.axis_index("c")
    sid = lax.axis_index("s")
    nblk = jnp.where(c == 0, GBLK0, GBLK1)
    base = blk_off + jnp.where(c == 0, sid * GBLK0,
                               NS * GBLK0 + sid * GBLK1)
    pltpu.sync_copy(j_hbm.at[pl.ds(base, GBLK1)],
                    idx_all.at[pl.ds(0, GBLK1)])

    @pl.when(c == 0)
    def _load_rest():
        pltpu.sync_copy(j_hbm.at[pl.ds(base + GBLK1, GBLK0 - GBLK1)],
                        idx_all.at[pl.ds(GBLK1, GBLK0 - GBLK1)])
    _gather_pass(phi_hbm, phij_hbm, idx_all, r0, r1, g0, g1, w0, w1, base,
                 nblk, blk_off)
    _inner_pass(v_hbm, vec_hbm, inner_hbm, idx_all, r0, r1, vb0, vb1, ibuf,
                g0, g1, w0, base, nblk, blk_off)


def _gather_sc(phi, v2d, j2d, vecT, chunk):
    mesh = plsc.VectorSubcoreMesh(core_axis_name="c", subcore_axis_name="s",
                                  num_cores=NC, num_subcores=NS)
    k = pl.kernel(
        functools.partial(_gather_body, chunk * CBLK),
        out_type=[
            jax.ShapeDtypeStruct((E_C, H3), jnp.float32),
            jax.ShapeDtypeStruct((E_C, H), jnp.float32),
        ],
        mesh=mesh,
        scratch_types=[
            pltpu.VMEM((GBLK0, EB), jnp.int32),
            pltpu.VMEM((EB, H3), jnp.float32),
            pltpu.VMEM((EB, H3), jnp.float32),
            pltpu.VMEM((3, EB), jnp.float32),
            pltpu.VMEM((3, EB), jnp.float32),
            pltpu.VMEM((EB, H), jnp.float32),
            pltpu.SemaphoreType.DMA,
            pltpu.SemaphoreType.DMA,
            pltpu.SemaphoreType.DMA,
            pltpu.SemaphoreType.DMA,
        ],
    )
    return k(phi, v2d, vecT, j2d)


def _edge_body(rbf_ref, wr_ref, br_ref, cut_ref, vx_ref, vy_ref, vz_ref,
               phij_ref, inner_ref, y_ref):
    w = (jnp.dot(rbf_ref[...], wr_ref[...],
                 preferred_element_type=jnp.float32) + br_ref[...]) * cut_ref[...]
    x = phij_ref[...] * w
    x_ss = x[:, 0:H]
    x_sv = x[:, H:2 * H]
    x_vv = x[:, 2 * H:3 * H]
    vx, vy, vz = vx_ref[...], vy_ref[...], vz_ref[...]
    coeff = x_sv + inner_ref[...] * x_vv
    y_ref[...] = jnp.concatenate(
        [x_ss, coeff * vx, coeff * vy, coeff * vz], axis=1)


def _edge_tc(rbf, Wr, br, cut, vx, vy, vz, phij, inner, chunk):
    blk = 512
    grid = (E_C // blk,)
    koff = chunk * (E_C // blk)
    return pl.pallas_call(
        _edge_body,
        grid=grid,
        in_specs=[
            pl.BlockSpec((blk, 20), lambda i: (i + koff, 0)),
            pl.BlockSpec((20, H3), lambda i: (0, 0)),
            pl.BlockSpec((1, H3), lambda i: (0, 0)),
            pl.BlockSpec((blk, 1), lambda i: (i + koff, 0)),
            pl.BlockSpec((blk, 1), lambda i: (i + koff, 0)),
            pl.BlockSpec((blk, 1), lambda i: (i + koff, 0)),
            pl.BlockSpec((blk, 1), lambda i: (i + koff, 0)),
            pl.BlockSpec((blk, H3), lambda i: (i, 0)),
            pl.BlockSpec((blk, H), lambda i: (i, 0)),
        ],
        out_specs=pl.BlockSpec((blk, 4 * H), lambda i: (i, 0)),
        out_shape=jax.ShapeDtypeStruct((E_C, 4 * H), jnp.float32),
    )(rbf, Wr, br, cut, vx, vy, vz, phij, inner)


def _scatter_body(blk_off, y_hbm, i_hbm, init_hbm, acc_hbm, acc_sh,
                  i0, i1, yb0, yb1, rs0, rs1, ss0, ss1):
    c = lax.axis_index("c")
    sid = lax.axis_index("s")
    r0 = sid * ROWS_PER_TILE
    base = sid * SCATTER_BLKS

    for g in range(2):
        grp = c * 2 + g
        pltpu.sync_copy(init_hbm.at[grp, pl.ds(r0, ROWS_PER_TILE)],
                        acc_sh.at[pl.ds(r0, ROWS_PER_TILE)])
        plsc.subcore_barrier()

        def rd(b, ib, yb, sem):
            pltpu.async_copy(i_hbm.at[blk_off + base + b], ib, sem)
            pltpu.async_copy(
                y_hbm.at[pl.ds((base + b) * EB, EB), pl.ds(grp * H, H)],
                yb, sem)

        def rd_wait(ib, yb, sem):
            pltpu.make_async_copy(i_hbm.at[blk_off + base], ib, sem).wait()
            pltpu.make_async_copy(y_hbm.at[pl.ds(base * EB, EB),
                                           pl.ds(grp * H, H)], yb, sem).wait()

        def sc(ib, yb, sem):
            pltpu.async_copy(yb, acc_sh.at[ib], sem, add=True)

        def sc_wait(ib, yb, sem):
            pltpu.make_async_copy(yb, acc_sh.at[ib], sem).wait()

        rd(0, i0, yb0, rs0)
        rd(1, i1, yb1, rs1)

        def body(i, carry):
            b = 2 * i
            rd_wait(i0, yb0, rs0)
            sc(i0, yb0, ss0)
            rd_wait(i1, yb1, rs1)
            sc(i1, yb1, ss1)
            sc_wait(i0, yb0, ss0)
            rd(b + 2, i0, yb0, rs0)
            sc_wait(i1, yb1, ss1)
            rd(b + 3, i1, yb1, rs1)
            return carry

        lax.fori_loop(0, SCATTER_BLKS // 2 - 1, body, 0)
        rd_wait(i0, yb0, rs0)
        sc(i0, yb0, ss0)
        rd_wait(i1, yb1, rs1)
        sc(i1, yb1, ss1)
        sc_wait(i0, yb0, ss0)
        sc_wait(i1, yb1, ss1)

        plsc.subcore_barrier()
        pltpu.sync_copy(acc_sh.at[pl.ds(r0, ROWS_PER_TILE)],
                        acc_hbm.at[grp, pl.ds(r0, ROWS_PER_TILE)])
        plsc.subcore_barrier()


def _scatter_sc(y, i2d, init, chunk):
    mesh = plsc.VectorSubcoreMesh(core_axis_name="c", subcore_axis_name="s",
                                  num_cores=NC, num_subcores=NS)
    k = pl.kernel(
        functools.partial(_scatter_body, chunk * CBLK),
        out_type=[jax.ShapeDtypeStruct((4, N_ACC, H), jnp.float32)],
        mesh=mesh,
        scratch_types=[
            pltpu.VMEM_SHARED((N_ACC, H), jnp.float32),
            pltpu.VMEM((EB,), jnp.int32),
            pltpu.VMEM((EB,), jnp.int32),
            pltpu.VMEM((EB, H), jnp.float32),
            pltpu.VMEM((EB, H), jnp.float32),
            pltpu.SemaphoreType.DMA,
            pltpu.SemaphoreType.DMA,
            pltpu.SemaphoreType.DMA,
            pltpu.SemaphoreType.DMA,
        ],
    )
    return k(y, i2d, init)[0]


def _update_body(s_ref, v_ref, acc_ref, wv_ref, wu1_ref, bu1_ref, wu2_ref,
                 bu2_ref, snew_ref, vnew_ref):
    s2 = s_ref[...] + acc_ref[0]
    v = v_ref[...]
    v2_0 = v[:, 0:H] + acc_ref[1]
    v2_1 = v[:, H:2 * H] + acc_ref[2]
    v2_2 = v[:, 2 * H:3 * H] + acc_ref[3]
    wv = wv_ref[...]
    vp0 = jnp.dot(v2_0, wv, preferred_element_type=jnp.float32)
    vp1 = jnp.dot(v2_1, wv, preferred_element_type=jnp.float32)
    vp2 = jnp.dot(v2_2, wv, preferred_element_type=jnp.float32)
    vu0, vv0 = vp0[:, 0:H], vp0[:, H:2 * H]
    vu1, vv1 = vp1[:, 0:H], vp1[:, H:2 * H]
    vu2, vv2 = vp2[:, 0:H], vp2[:, H:2 * H]
    norm = jnp.sqrt(vv0 * vv0 + vv1 * vv1 + vv2 * vv2)
    s_in = jnp.concatenate([s2, norm], axis=1)
    hid = jax.nn.silu(jnp.dot(s_in, wu1_ref[...],
                              preferred_element_type=jnp.float32) + bu1_ref[...])
    s_out = jnp.dot(hid, wu2_ref[...],
                    preferred_element_type=jnp.float32) + bu2_ref[...]
    a_ss = s_out[:, 0:H]
    a_sv = s_out[:, H:2 * H]
    a_vv = s_out[:, 2 * H:3 * H]
    snew_ref[...] = s2 + a_ss
    vnew_ref[...] = jnp.concatenate([
        v2_0 + a_vv * vu0 + a_sv * vv0,
        v2_1 + a_vv * vu1 + a_sv * vv1,
        v2_2 + a_vv * vu2 + a_sv * vv2,
    ], axis=1)


def _update_tc(s, v2d, acc, Wv, Wu1, bu1, Wu2, bu2):
    blk = 1000 if N_NODES % 1000 == 0 else N_NODES
    grid = (N_NODES // blk,)
    return pl.pallas_call(
        _update_body,
        grid=grid,
        in_specs=[
            pl.BlockSpec((blk, H), lambda i: (i, 0)),
            pl.BlockSpec((blk, H3), lambda i: (i, 0)),
            pl.BlockSpec((4, blk, H), lambda i: (0, i, 0)),
            pl.BlockSpec((H, 2 * H), lambda i: (0, 0)),
            pl.BlockSpec((2 * H, H), lambda i: (0, 0)),
            pl.BlockSpec((1, H), lambda i: (0, 0)),
            pl.BlockSpec((H, H3), lambda i: (0, 0)),
            pl.BlockSpec((1, H3), lambda i: (0, 0)),
        ],
        out_specs=[
            pl.BlockSpec((blk, H), lambda i: (i, 0)),
            pl.BlockSpec((blk, H3), lambda i: (i, 0)),
        ],
        out_shape=[
            jax.ShapeDtypeStruct((N_NODES, H), jnp.float32),
            jax.ShapeDtypeStruct((N_NODES, H3), jnp.float32),
        ],
    )(s, v2d, acc, Wv, Wu1, bu1, Wu2, bu2)


def kernel(s, v, edge_index, edge_rbf, edge_cutoff, edge_vec,
           W1, b1, W2, b2, Wr, br, Wv, Wu1, bu1, Wu2, bu2):
    pad = E_PAD - N_EDGES
    j_idx = edge_index[1].astype(jnp.int32)
    i_idx = edge_index[0].astype(jnp.int32)
    j2d = jnp.pad(j_idx, (0, pad)).reshape(NBLK, EB)
    i2d = jnp.pad(i_idx, (0, pad)).reshape(NBLK, EB)
    rbf_p = jnp.pad(edge_rbf, ((0, pad), (0, 0)))
    cut_p = jnp.pad(edge_cutoff, (0, pad)).reshape(E_PAD, 1)
    vec_p = jnp.pad(edge_vec, ((0, pad), (0, 0)))
    vx = vec_p[:, 0:1]
    vy = vec_p[:, 1:2]
    vz = vec_p[:, 2:3]
    v2d = v.reshape(N_NODES, H3)

    vecT = vec_p.T
    phi = _phi_tc(s, W1, b1.reshape(1, H), W2, b2.reshape(1, H3))
    acc = jnp.zeros((4, N_ACC, H), jnp.float32)
    for k in range(NCHUNK):
        phij, inner = _gather_sc(phi, v2d, j2d, vecT, k)
        y = _edge_tc(rbf_p, Wr, br.reshape(1, H3), cut_p, vx, vy, vz, phij,
                     inner, k)
        acc = _scatter_sc(y, i2d, acc, k)
    s_new, vnew2d = _update_tc(s, v2d, acc, Wv, Wu1, bu1.reshape(1, H),
                               Wu2, bu2.reshape(1, H3))
    return (s_new, vnew2d.reshape(N_NODES, 3, H))
